# baseline (speedup 1.0000x reference)
import numpy as np
import jax
import jax.numpy as jnp
from jax.experimental import pallas as pl
from jax.experimental.pallas import tpu as pltpu

_M = 2
_K = 4
_L = 3
_D_OUT = 8
_POOL = 2
_TEMP = 0.5
_TWO_K = 1 << _K
_NUM_LAYERS = 4
_C_IN0 = 4
_TARGET_LANES = 65536


def _fern_idx():
    rng = np.random.RandomState(0)
    layers = []
    c_in = _C_IN0
    for _ in range(_NUM_LAYERS):
        idx = np.stack(
            [rng.randint(0, c_in, (_M, _K)),
             rng.randint(0, c_in, (_M, _K)),
             rng.randint(0, _L, (_M, _K)),
             rng.randint(0, _L, (_M, _K)),
             rng.randint(0, _L, (_M, _K)),
             rng.randint(0, _L, (_M, _K))],
            axis=-1)
        layers.append(tuple(tuple(tuple(int(v) for v in idx[m, k])
                                  for k in range(_K)) for m in range(_M)))
        c_in = _D_OUT
    return tuple(layers)


_IDX_LAYERS = _fern_idx()


def _build_body(idx_layers, w0, hw):
    half_inv_temp = 0.5 / _TEMP

    def rolled_slab(slab, off):
        if off % hw == 0:
            return slab
        return pltpu.roll(slab, hw - (off % hw), axis=1)

    shrink = _NUM_LAYERS * ((_L - 1) + (_POOL - 1))
    hf = hw // w0 - shrink
    wf = w0 - shrink
    hw_keep = hf * w0
    n_valid = hf * wf

    def body(x0_ref, thr_ref, tbl_ref, out_ref, xbuf):
        x0 = jnp.transpose(x0_ref[...], (1, 0, 2))
        for li in range(_NUM_LAYERS):
            idx = idx_layers[li]
            src = x0 if li == 0 else xbuf[...]

            picks = {}
            for m in range(_M):
                for k in range(_K):
                    c1, c2, dy1, dx1, dy2, dx2 = idx[m][k]
                    for c, o in ((c1, dy1 * w0 + dx1), (c2, dy2 * w0 + dx2)):
                        if (c, o) not in picks:
                            picks[(c, o)] = rolled_slab(src[c], o)

            rows = []
            for m in range(_M):
                for k in range(_K):
                    c1, c2, dy1, dx1, dy2, dx2 = idx[m][k]
                    r = (li * _M + m) * _K + k
                    p1 = picks[(c1, dy1 * w0 + dx1)]
                    p2 = picks[(c2, dy2 * w0 + dx2)]
                    rows.append((p1 - p2 - thr_ref[r]) * half_inv_temp)
            logits = jnp.stack(rows, axis=0)
            b = 0.5 * jnp.tanh(logits) + 0.5
            bn = 1.0 - b

            nb_i = b.shape[1]
            npb = nb_i * hw
            p8s, b3rows = [], []
            for m in range(_M):
                r0 = m * _K
                p = jnp.concatenate([bn[r0:r0 + 1], b[r0:r0 + 1]], axis=0)
                for k in (1, 2):
                    p = jnp.concatenate([p * bn[r0 + k:r0 + k + 1],
                                         p * b[r0 + k:r0 + k + 1]], axis=0)
                p8s.append(p.astype(jnp.bfloat16).reshape(_TWO_K // 2, npb))
                b3rows.append(b[r0 + 3:r0 + 4].reshape(1, npb))
            half = jnp.concatenate(p8s, axis=0)
            tblp = tbl_ref[pl.ds(li * 2 * _M * _D_OUT, 2 * _M * _D_OUT), :]
            v4 = jnp.dot(tblp, half,
                         preferred_element_type=jnp.float32)
            votes = None
            for m in range(_M):
                lo = v4[(2 * m) * _D_OUT:(2 * m + 1) * _D_OUT]
                dl = v4[(2 * m + 1) * _D_OUT:(2 * m + 2) * _D_OUT]
                vm = lo + b3rows[m] * dl
                votes = vm if votes is None else votes + vm

            def roll2(v, off):
                sz = v.shape[-1]
                return pltpu.roll(v, sz - (off % sz), axis=1)
            acc = votes + roll2(votes, 1)
            acc = acc + roll2(acc, w0)
            stacked = acc.reshape(_D_OUT, nb_i, hw)
            if li == _NUM_LAYERS - 1:
                v = stacked[:, :, :hw_keep]
                lanes = hw_keep
                iota = jax.lax.broadcasted_iota(jnp.int32, (1, 1, lanes), 2)
                p, k = w0, wf
                while p < lanes:
                    pull = p - k
                    shifted = pltpu.roll(v, lanes - pull, axis=2)
                    v = jnp.where((iota % (2 * p)) < k, v, shifted)
                    p, k = 2 * p, 2 * k
                u = jnp.transpose(v[:, :, :n_valid], (1, 0, 2))
                out_ref[...] = u.reshape(u.shape[0], _D_OUT * n_valid)
            else:
                xbuf[...] = stacked

    return body


def kernel(x_nchw, thr, table):
    n, c0, h0, w0 = x_nchw.shape
    hw = h0 * w0
    nb = 1
    for d in range(1, n + 1):
        if n % d == 0 and d * hw <= _TARGET_LANES:
            nb = d
    t = n // nb

    x = x_nchw.astype(jnp.float32).reshape(n, c0, hw)

    tb = table.reshape(_NUM_LAYERS, _D_OUT, _M, 2, _TWO_K // 2)
    t5 = jnp.transpose(tb, (0, 2, 3, 1, 4))
    t5 = jnp.stack([t5[:, :, 0], t5[:, :, 1] - t5[:, :, 0]], axis=2)
    eye = jnp.eye(_M, dtype=table.dtype)
    tp = t5[:, :, :, :, None, :] * eye[None, :, None, None, :, None]
    tp = tp.reshape(_NUM_LAYERS * 2 * _M * _D_OUT, _M * (_TWO_K // 2))
    tp = (tp * (1.0 / (_POOL * _POOL))).astype(jnp.bfloat16)

    shrink = _NUM_LAYERS * ((_L - 1) + (_POOL - 1))
    hf, wf = h0 - shrink, w0 - shrink
    hw_keep = hf * w0

    body = _build_body(_IDX_LAYERS, w0, hw)
    out = pl.pallas_call(
        body,
        out_shape=jax.ShapeDtypeStruct((n, _D_OUT * hf * wf), jnp.float32),
        grid=(t,),
        in_specs=[
            pl.BlockSpec((nb, c0, hw), lambda i: (i, 0, 0)),
            pl.BlockSpec(memory_space=pltpu.MemorySpace.SMEM),
            pl.BlockSpec((_NUM_LAYERS * 2 * _M * _D_OUT, _M * (_TWO_K // 2)),
                         lambda i: (0, 0)),
        ],
        out_specs=pl.BlockSpec((nb, _D_OUT * hf * wf), lambda i: (i, 0)),
        scratch_shapes=[pltpu.VMEM((_D_OUT, nb, hw), jnp.float32)],
        compiler_params=pltpu.CompilerParams(
            dimension_semantics=("parallel",)),
    )(x, thr, tp)

    return out


# speedup vs baseline: 1.0275x; 1.0275x over previous
import numpy as np
import jax
import jax.numpy as jnp
from jax.experimental import pallas as pl
from jax.experimental.pallas import tpu as pltpu

_M = 2
_K = 4
_L = 3
_D_OUT = 8
_POOL = 2
_TEMP = 0.5
_TWO_K = 1 << _K
_NUM_LAYERS = 4
_C_IN0 = 4
_TARGET_LANES = 32768


def _fern_idx():
    rng = np.random.RandomState(0)
    layers = []
    c_in = _C_IN0
    for _ in range(_NUM_LAYERS):
        idx = np.stack(
            [rng.randint(0, c_in, (_M, _K)),
             rng.randint(0, c_in, (_M, _K)),
             rng.randint(0, _L, (_M, _K)),
             rng.randint(0, _L, (_M, _K)),
             rng.randint(0, _L, (_M, _K)),
             rng.randint(0, _L, (_M, _K))],
            axis=-1)
        layers.append(tuple(tuple(tuple(int(v) for v in idx[m, k])
                                  for k in range(_K)) for m in range(_M)))
        c_in = _D_OUT
    return tuple(layers)


_IDX_LAYERS = _fern_idx()


def _build_body(idx_layers, w0, hw):
    half_inv_temp = 0.5 / _TEMP

    def rolled_slab(slab, off):
        if off % hw == 0:
            return slab
        return pltpu.roll(slab, hw - (off % hw), axis=1)

    shrink = _NUM_LAYERS * ((_L - 1) + (_POOL - 1))
    hf = hw // w0 - shrink
    wf = w0 - shrink
    hw_keep = hf * w0
    n_valid = hf * wf

    def body(x0_ref, thr_ref, tbl_ref, out_ref, xbuf):
        x0 = jnp.transpose(x0_ref[...], (1, 0, 2))
        for li in range(_NUM_LAYERS):
            idx = idx_layers[li]
            src = x0 if li == 0 else xbuf[...]

            picks = {}
            for m in range(_M):
                for k in range(_K):
                    c1, c2, dy1, dx1, dy2, dx2 = idx[m][k]
                    for c, o in ((c1, dy1 * w0 + dx1), (c2, dy2 * w0 + dx2)):
                        if (c, o) not in picks:
                            picks[(c, o)] = rolled_slab(src[c], o)

            rows = []
            for m in range(_M):
                for k in range(_K):
                    c1, c2, dy1, dx1, dy2, dx2 = idx[m][k]
                    r = (li * _M + m) * _K + k
                    p1 = picks[(c1, dy1 * w0 + dx1)]
                    p2 = picks[(c2, dy2 * w0 + dx2)]
                    rows.append((p1 - p2 - thr_ref[r]) * half_inv_temp)
            logits = jnp.stack(rows, axis=0)
            b = 0.5 * jnp.tanh(logits) + 0.5
            bn = 1.0 - b

            nb_i = b.shape[1]
            npb = nb_i * hw
            p8s, b3rows = [], []
            for m in range(_M):
                r0 = m * _K
                p = jnp.concatenate([bn[r0:r0 + 1], b[r0:r0 + 1]], axis=0)
                for k in (1, 2):
                    p = jnp.concatenate([p * bn[r0 + k:r0 + k + 1],
                                         p * b[r0 + k:r0 + k + 1]], axis=0)
                p8s.append(p.astype(jnp.bfloat16).reshape(_TWO_K // 2, npb))
                b3rows.append(b[r0 + 3:r0 + 4].reshape(1, npb))
            half = jnp.concatenate(p8s, axis=0)
            tblp = tbl_ref[pl.ds(li * 2 * _M * _D_OUT, 2 * _M * _D_OUT), :]
            v4 = jnp.dot(tblp, half,
                         preferred_element_type=jnp.float32)
            votes = None
            for m in range(_M):
                lo = v4[(2 * m) * _D_OUT:(2 * m + 1) * _D_OUT]
                dl = v4[(2 * m + 1) * _D_OUT:(2 * m + 2) * _D_OUT]
                vm = lo + b3rows[m] * dl
                votes = vm if votes is None else votes + vm

            def roll2(v, off):
                sz = v.shape[-1]
                return pltpu.roll(v, sz - (off % sz), axis=1)
            acc = votes + roll2(votes, 1)
            acc = acc + roll2(acc, w0)
            stacked = acc.reshape(_D_OUT, nb_i, hw)
            if li == _NUM_LAYERS - 1:
                v = stacked[:, :, :hw_keep]
                lanes = hw_keep
                iota = jax.lax.broadcasted_iota(jnp.int32, (1, 1, lanes), 2)
                p, k = w0, wf
                while p < lanes:
                    pull = p - k
                    shifted = pltpu.roll(v, lanes - pull, axis=2)
                    v = jnp.where((iota % (2 * p)) < k, v, shifted)
                    p, k = 2 * p, 2 * k
                u = jnp.transpose(v[:, :, :n_valid], (1, 0, 2))
                out_ref[...] = u.reshape(u.shape[0], _D_OUT * n_valid)
            else:
                xbuf[...] = stacked

    return body


def kernel(x_nchw, thr, table):
    n, c0, h0, w0 = x_nchw.shape
    hw = h0 * w0
    nb = 1
    for d in range(1, n + 1):
        if n % d == 0 and d * hw <= _TARGET_LANES:
            nb = d
    t = n // nb

    x = x_nchw.astype(jnp.float32).reshape(n, c0, hw)

    tb = table.reshape(_NUM_LAYERS, _D_OUT, _M, 2, _TWO_K // 2)
    t5 = jnp.transpose(tb, (0, 2, 3, 1, 4))
    t5 = jnp.stack([t5[:, :, 0], t5[:, :, 1] - t5[:, :, 0]], axis=2)
    eye = jnp.eye(_M, dtype=table.dtype)
    tp = t5[:, :, :, :, None, :] * eye[None, :, None, None, :, None]
    tp = tp.reshape(_NUM_LAYERS * 2 * _M * _D_OUT, _M * (_TWO_K // 2))
    tp = (tp * (1.0 / (_POOL * _POOL))).astype(jnp.bfloat16)

    shrink = _NUM_LAYERS * ((_L - 1) + (_POOL - 1))
    hf, wf = h0 - shrink, w0 - shrink
    hw_keep = hf * w0

    body = _build_body(_IDX_LAYERS, w0, hw)
    out = pl.pallas_call(
        body,
        out_shape=jax.ShapeDtypeStruct((n, _D_OUT * hf * wf), jnp.float32),
        grid=(t,),
        in_specs=[
            pl.BlockSpec((nb, c0, hw), lambda i: (i, 0, 0)),
            pl.BlockSpec(memory_space=pltpu.MemorySpace.SMEM),
            pl.BlockSpec((_NUM_LAYERS * 2 * _M * _D_OUT, _M * (_TWO_K // 2)),
                         lambda i: (0, 0)),
        ],
        out_specs=pl.BlockSpec((nb, _D_OUT * hf * wf), lambda i: (i, 0)),
        scratch_shapes=[pltpu.VMEM((_D_OUT, nb, hw), jnp.float32)],
        compiler_params=pltpu.CompilerParams(
            dimension_semantics=("parallel",)),
    )(x, thr, tp)

    return out
